# Initial kernel scaffold; baseline (speedup 1.0000x reference)
import jax
import jax.numpy as jnp
from jax.experimental import pallas as pl


def kernel(output_heatmap, output_bbox, output_offset, image_id):
    raise NotImplementedError("write your pallas kernel here")


# speedup vs baseline: 2.9834x; 2.9834x over previous
import functools

import jax
import jax.numpy as jnp
from jax import lax
from jax.experimental import pallas as pl
from jax.experimental.pallas import tpu as pltpu
from jax.experimental.pallas import tpu_sc as plsc

B, H, W = 16, 128, 128
HW = H * W
K = 100
DR = 4.0
THR = 0.1
NCHUNKS = HW // 16

HIST_BASE = 63280
HIST_N = 1760
FIN_CAP = 272
SEL = 112
NEG_INF = float("-inf")


def _body(heat_hbm, off_hbm, wh_hbm, img_hbm, out_hbm,
          heat_pad, rmax_pad, cand_s, cand_i, hist,
          fin_s, fin_i, sel_s, sel_i,
          idx_ox, idx_oy, idx_bw, idx_bh,
          val_ox, val_oy, val_bw, val_bh,
          det_v, img_v, sem):
  wid = lax.axis_index("s") * 2 + lax.axis_index("c")

  @pl.when(wid < B)
  def _():
    b = wid
    iota = lax.iota(jnp.int32, 16)
    zeros_i = jnp.zeros((16,), jnp.int32)
    ones_i = jnp.ones((16,), jnp.int32)
    zeros_f = jnp.zeros((16,), jnp.float32)
    ninf = jnp.full((16,), NEG_INF, jnp.float32)

    pltpu.sync_copy(heat_hbm.at[pl.ds(b * HW, HW)], heat_pad.at[pl.ds(16, HW)])
    pltpu.sync_copy(img_hbm, img_v)

    for g in range(8):
      rmax_pad[pl.ds(g * 16, 16)] = ninf
      rmax_pad[pl.ds(W + HW + g * 16, 16)] = ninf
    for g in range(HIST_N // 16):
      hist[pl.ds(g * 16, 16)] = zeros_i
    for g in range(704 // 16):
      det_v[pl.ds(g * 16, 16)] = zeros_f
    for g in range(FIN_CAP // 16):
      fin_s[pl.ds(g * 16, 16)] = jnp.full((16,), -1.0, jnp.float32)
    for g in range(SEL // 16):
      sel_i[pl.ds(g * 16, 16)] = zeros_i
      sel_s[pl.ds(g * 16, 16)] = zeros_f

    def row_pass(c, _):
      base = 16 + c * 16
      xc = (c % 8) * 16 + iota
      mid = heat_pad[pl.ds(base, 16)]
      left = heat_pad[pl.ds(base - 1, 16)]
      right = heat_pad[pl.ds(base + 1, 16)]
      left = jnp.where(xc >= 1, left, ninf)
      right = jnp.where(xc <= W - 2, right, ninf)
      rmax_pad[pl.ds(W + c * 16, 16)] = jnp.maximum(mid, jnp.maximum(left, right))
      return 0
    lax.fori_loop(0, NCHUNKS, row_pass, 0)

    thr = jnp.full((16,), THR, jnp.float32)

    def col_pass(c, n_splat):
      base = W + c * 16
      mid = rmax_pad[pl.ds(base, 16)]
      top = rmax_pad[pl.ds(base - W, 16)]
      bot = rmax_pad[pl.ds(base + W, 16)]
      hmax = jnp.maximum(mid, jnp.maximum(top, bot))
      h = heat_pad[pl.ds(16 + c * 16, 16)]
      keep = (h == hmax) & (h > thr)
      prefix = plsc.cumsum(jnp.where(keep, ones_i, zeros_i))
      pos = n_splat + prefix - 1
      plsc.store_scatter(cand_s, [pos], h, mask=keep)
      plsc.store_scatter(cand_i, [pos], c * 16 + iota, mask=keep)
      return n_splat + plsc.all_reduce_population_count(keep)
    n_splat = lax.fori_loop(0, NCHUNKS, col_pass, zeros_i)
    n = lax.reduce_max_p.bind(n_splat, axes=(0,))

    def hist_pass(c, _):
      base = c * 16
      valid = (base + iota) < n_splat
      s = cand_s[pl.ds(base, 16)]
      bucket = lax.shift_right_logical(plsc.bitcast(s, jnp.int32), 14) - HIST_BASE
      plsc.addupdate_scatter(hist, [bucket], ones_i, mask=valid)
      return 0
    lax.fori_loop(0, (n + 15) // 16, hist_pass, 0)

    def scan_cond(state):
      v, total, _ = state
      return (v >= 0) & (total < K)

    def scan_step(state):
      v, total, blo = state
      chunk = hist[pl.ds(v * 16, 16)]
      ssum = lax.reduce_sum_p.bind(chunk, axes=(0,))

      def boundary():
        suffix = lax.rev(plsc.cumsum(lax.rev(chunk, (0,))), (0,))
        above = total + suffix - chunk
        mk = (above < K) & ((above + chunk) >= K)
        lane = lax.reduce_max_p.bind(jnp.where(mk, iota, zeros_i), axes=(0,))
        return (HIST_BASE + v * 16 + lane) << 14

      blo_new = lax.cond(total + ssum >= K, boundary, lambda: blo)
      return v - 1, total + ssum, blo_new

    _, _, blo_bits = lax.while_loop(
        scan_cond, scan_step, (HIST_N // 16 - 1, jnp.int32(0), jnp.int32(0)))
    blo = plsc.bitcast(jnp.full((16,), blo_bits, jnp.int32), jnp.float32)

    def collect(c, m_splat):
      base = c * 16
      valid = (base + iota) < n_splat
      s = cand_s[pl.ds(base, 16)]
      ii = cand_i[pl.ds(base, 16)]
      mk = valid & (s >= blo)
      prefix = plsc.cumsum(jnp.where(mk, ones_i, zeros_i))
      pos = m_splat + prefix - 1
      mk = mk & (pos < FIN_CAP)
      plsc.store_scatter(fin_s, [pos], s, mask=mk)
      plsc.store_scatter(fin_i, [pos], ii, mask=mk)
      return m_splat + plsc.all_reduce_population_count(mk)
    m_splat = lax.fori_loop(0, (n + 15) // 16, collect, zeros_i)
    m = lax.reduce_max_p.bind(m_splat, axes=(0,))
    mc = (m + 15) // 16
    nsel = jnp.minimum(m_splat, jnp.full((16,), K, jnp.int32))

    lane0 = iota == 0

    def rank_one(e, _):
      ev = jnp.full((16,), e, jnp.int32)
      se = plsc.load_gather(fin_s, [ev])
      ie = plsc.load_gather(fin_i, [ev])

      def cnt(c, r):
        fs = fin_s[pl.ds(c * 16, 16)]
        fi = fin_i[pl.ds(c * 16, 16)]
        beats = (fs > se) | ((fs == se) & (fi < ie))
        return r + plsc.all_reduce_population_count(beats)
      rank = lax.fori_loop(0, mc, cnt, zeros_i)
      mk = lane0 & (rank < K)
      plsc.store_scatter(sel_s, [rank], se, mask=mk)
      plsc.store_scatter(sel_i, [rank], ie, mask=mk)
      return 0
    lax.fori_loop(0, m, rank_one, 0)

    base_ox = b * 2 * HW
    for r in range(SEL // 16):
      si = sel_i[pl.ds(r * 16, 16)]
      idx_ox[pl.ds(r * 16, 16)] = si + base_ox
      idx_oy[pl.ds(r * 16, 16)] = si + (base_ox + HW)
      idx_bw[pl.ds(r * 16, 16)] = si + base_ox
      idx_bh[pl.ds(r * 16, 16)] = si + (base_ox + HW)

    c1 = pltpu.make_async_copy(off_hbm.at[idx_ox], val_ox, sem)
    c2 = pltpu.make_async_copy(off_hbm.at[idx_oy], val_oy, sem)
    c3 = pltpu.make_async_copy(wh_hbm.at[idx_bw], val_bw, sem)
    c4 = pltpu.make_async_copy(wh_hbm.at[idx_bh], val_bh, sem)
    c1.start(); c2.start(); c3.start(); c4.start()
    c1.wait(); c2.wait(); c3.wait(); c4.wait()

    img = plsc.load_gather(img_v, [jnp.full((16,), b, jnp.int32)])
    for r in range(SEL // 16):
      rv = r * 16 + iota
      mk = rv < nsel
      s = sel_s[pl.ds(r * 16, 16)]
      si = sel_i[pl.ds(r * 16, 16)]
      ox = val_ox[pl.ds(r * 16, 16)]
      oy = val_oy[pl.ds(r * 16, 16)]
      bw = val_bw[pl.ds(r * 16, 16)]
      bh = val_bh[pl.ds(r * 16, 16)]
      xs = lax.convert_element_type(si & (W - 1), jnp.float32) + ox
      ys = lax.convert_element_type(lax.shift_right_logical(si, 7), jnp.float32) + oy
      bw2 = bw * 0.5
      bh2 = bh * 0.5
      p7 = rv * 7
      plsc.store_scatter(det_v, [p7], img, mask=mk)
      plsc.store_scatter(det_v, [p7 + 1], (xs - bw2) * DR, mask=mk)
      plsc.store_scatter(det_v, [p7 + 2], (ys - bh2) * DR, mask=mk)
      plsc.store_scatter(det_v, [p7 + 3], (xs + bw2) * DR, mask=mk)
      plsc.store_scatter(det_v, [p7 + 4], (ys + bh2) * DR, mask=mk)
      plsc.store_scatter(det_v, [p7 + 5], s, mask=mk)

    pltpu.sync_copy(det_v, out_hbm.at[pl.ds(b * 704, 704)])


@jax.jit
def kernel(output_heatmap, output_bbox, output_offset, image_id):
  heat = output_heatmap.reshape(B * HW)
  off = output_offset.reshape(B * 2 * HW)
  wh = output_bbox.reshape(B * 2 * HW)
  img = image_id.astype(jnp.float32)

  mesh = plsc.VectorSubcoreMesh(
      core_axis_name="c", subcore_axis_name="s", num_cores=2, num_subcores=16)
  f = pl.kernel(
      _body,
      out_type=jax.ShapeDtypeStruct((B * 704,), jnp.float32),
      mesh=mesh,
      compiler_params=pltpu.CompilerParams(needs_layout_passes=False),
      scratch_types=[
          pltpu.VMEM((16 + HW + 16,), jnp.float32),
          pltpu.VMEM((W + HW + W,), jnp.float32),
          pltpu.VMEM((HW + 16,), jnp.float32),
          pltpu.VMEM((HW + 16,), jnp.int32),
          pltpu.VMEM((HIST_N,), jnp.int32),
          pltpu.VMEM((FIN_CAP + 16,), jnp.float32),
          pltpu.VMEM((FIN_CAP + 16,), jnp.int32),
          pltpu.VMEM((SEL,), jnp.float32),
          pltpu.VMEM((SEL,), jnp.int32),
          pltpu.VMEM((SEL,), jnp.int32),
          pltpu.VMEM((SEL,), jnp.int32),
          pltpu.VMEM((SEL,), jnp.int32),
          pltpu.VMEM((SEL,), jnp.int32),
          pltpu.VMEM((SEL,), jnp.float32),
          pltpu.VMEM((SEL,), jnp.float32),
          pltpu.VMEM((SEL,), jnp.float32),
          pltpu.VMEM((SEL,), jnp.float32),
          pltpu.VMEM((704,), jnp.float32),
          pltpu.VMEM((16,), jnp.float32),
          pltpu.SemaphoreType.DMA,
      ],
  )
  out = f(heat, off, wh, img)
  return out.reshape(B, 704)[:, :700].reshape(B, K, 7)
